# baseline (speedup 1.0000x reference)
import functools

import numpy as np

import jax
import jax.numpy as jnp
from jax import lax
from jax.experimental import pallas as pl
from jax.experimental.pallas import tpu as pltpu
from jax.experimental.pallas import tpu_sc as plsc

H = 128
NC = 2
NSUB = 16
NW = NC * NSUB


def _sc_mesh():
    return plsc.VectorSubcoreMesh(core_axis_name="c", subcore_axis_name="s",
                                  num_cores=NC, num_subcores=NSUB)


@functools.lru_cache(maxsize=None)
def _sc_gather_add(E, N):
    EPW = E // NW
    C = 40
    NCH = EPW // C
    NBUF = 3
    SCH = 80
    NSTG = N // SCH
    assert EPW % C == 0 and N % SCH == 0

    @functools.partial(
        pl.kernel,
        out_type=jax.ShapeDtypeStruct((E, H // 2), jnp.int32),
        mesh=_sc_mesh(),
        scratch_types=[
            [pltpu.VMEM((C,), jnp.int32)] * NBUF,
            [pltpu.VMEM((C,), jnp.int32)] * NBUF,
            [pltpu.VMEM((C, H), jnp.float32)] * NBUF,
            [pltpu.VMEM((C, H), jnp.float32)] * NBUF,
            [pltpu.VMEM((C, H // 2), jnp.int32)] * NBUF,
            pltpu.VMEM((H,), jnp.float32),
            pltpu.VMEM_SHARED((N, H), jnp.float32),
            [pltpu.SemaphoreType.DMA] * NBUF,
            [pltpu.SemaphoreType.DMA] * NBUF,
            [pltpu.SemaphoreType.DMA] * NBUF,
            [pltpu.SemaphoreType.DMA] * NBUF,
            pltpu.SemaphoreType.DMA,
        ],
    )
    def k(hd_hbm, hs_hbm, dst_hbm, src_hbm, b1_hbm, out_hbm,
          idxl, idxr, rl, rr, tb, b1v, shared, gl, gr, osem, isem, ssem):
        c = lax.axis_index("c")
        s = lax.axis_index("s")
        base = (c * NSUB + s) * EPW
        pltpu.sync_copy(b1_hbm, b1v)

        def stage(tab_hbm):
            def body(kk, _):
                ch = s + kk * NSUB

                @pl.when(ch < NSTG)
                def _():
                    sl = pl.ds(ch * SCH, SCH)
                    pltpu.async_copy(tab_hbm.at[sl], shared.at[sl], ssem).wait()
                return 0

            lax.fori_loop(0, (NSTG + NSUB - 1) // NSUB, body, 0)

        def run(lidx_hbm, rem_hbm, ridx_hbm):
            def fetch_idx(b, ci):
                sl = pl.ds(base + ci * C, C)
                pltpu.async_copy(lidx_hbm.at[sl], idxl[b], isem[b])
                pltpu.async_copy(ridx_hbm.at[sl], idxr[b], isem[b])

            def wait_idx(b, ci):
                sl = pl.ds(base + ci * C, C)
                pltpu.make_async_copy(lidx_hbm.at[sl], idxl[b], isem[b]).wait()
                pltpu.make_async_copy(ridx_hbm.at[sl], idxr[b], isem[b]).wait()

            def fire(b):
                pltpu.async_copy(shared.at[idxl[b]], rl[b], gl[b])
                pltpu.async_copy(rem_hbm.at[idxr[b]], rr[b], gr[b])

            def wait_gather(b):
                pltpu.make_async_copy(shared.at[idxl[b]], rl[b], gl[b]).wait()
                pltpu.make_async_copy(rem_hbm.at[idxr[b]], rr[b], gr[b]).wait()

            def wait_out(b, ci):
                sl = pl.ds(base + ci * C, C)
                pltpu.make_async_copy(tb[b], out_hbm.at[sl], osem[b]).wait()

            def rhu_pack(x, y):
                ux = lax.bitcast_convert_type(x, jnp.uint32)
                uy = lax.bitcast_convert_type(y, jnp.uint32)
                w = (((ux + jnp.uint32(0x8000)) >> 16)
                     | ((uy + jnp.uint32(0x8000)) & jnp.uint32(0xFFFF0000)))
                return lax.bitcast_convert_type(w, jnp.int32)

            bias = [b1v[pl.ds(16 * j, 16)] for j in range(H // 16)]

            def consume(b, ci, steady=True):
                wait_gather(b)
                if steady:
                    @pl.when(ci + NBUF < NCH)
                    def _():
                        fetch_idx(b, ci + NBUF)

                @pl.when(ci >= NBUF)
                def _():
                    wait_out(b, ci - NBUF)

                def addrow(i, _):
                    for j in range(H // 32):
                        slx = pl.ds(32 * j, 16)
                        sly = pl.ds(32 * j + 16, 16)
                        x = jnp.maximum(rl[b][i, slx] + rr[b][i, slx]
                                        + bias[2 * j], 0.0)
                        y = jnp.maximum(rl[b][i, sly] + rr[b][i, sly]
                                        + bias[2 * j + 1], 0.0)
                        tb[b][i, pl.ds(16 * j, 16)] = rhu_pack(x, y)
                    return 0

                lax.fori_loop(0, C, addrow, 0)
                pltpu.async_copy(tb[b], out_hbm.at[pl.ds(base + ci * C, C)],
                                 osem[b])
                if steady:
                    @pl.when(ci + NBUF < NCH)
                    def _():
                        wait_idx(b, ci + NBUF)
                        fire(b)

            for b in range(NBUF):
                fetch_idx(b, b)
                wait_idx(b, b)
                fire(b)

            def step(p, _):
                for b in range(NBUF):
                    consume(b, NBUF * p + b)
                return 0

            lax.fori_loop(0, NCH // NBUF, step, 0)
            for ci in range((NCH // NBUF) * NBUF, NCH):
                consume(ci % NBUF, ci, steady=False)
            for ci in range(NCH - NBUF, NCH):
                wait_out(ci % NBUF, ci)

        @pl.when(c == 0)
        def _():
            stage(hd_hbm)

        @pl.when(c == 1)
        def _():
            stage(hs_hbm)

        plsc.subcore_barrier()

        @pl.when(c == 0)
        def _():
            run(dst_hbm, hs_hbm, src_hbm)

        @pl.when(c == 1)
        def _():
            run(src_hbm, hd_hbm, dst_hbm)

    return k


@functools.lru_cache(maxsize=None)
def _sc_scatter_add(E, NPAD):
    EPW = E // NW
    C = 40
    RPT = NPAD // NSUB
    ZC = 40
    OC = 128
    assert EPW % C == 0 and RPT % ZC == 0 and RPT % OC == 0 and RPT % 8 == 0

    @functools.partial(
        pl.kernel,
        out_type=jax.ShapeDtypeStruct((NC, NPAD, H), jnp.float32),
        mesh=_sc_mesh(),
        scratch_types=[
            [pltpu.VMEM((C,), jnp.int32)] * 2,
            [pltpu.VMEM((C, H), jnp.float32)] * 2,
            pltpu.VMEM((ZC, H), jnp.float32),
            pltpu.VMEM((OC, H), jnp.float32),
            pltpu.VMEM_SHARED((NPAD, H), jnp.float32),
            [pltpu.SemaphoreType.DMA] * 2,
            [pltpu.SemaphoreType.DMA] * 2,
        ],
    )
    def k(m_hbm, dst_hbm, out_hbm, idx, rows, zb, ob, shared, msem, isem):
        c = lax.axis_index("c")
        s = lax.axis_index("s")
        wid = c * NSUB + s
        base = wid * EPW
        NCH = EPW // C

        def fetch(b, ci):
            sl = pl.ds(base + ci * C, C)
            pltpu.async_copy(dst_hbm.at[sl], idx[b], isem[b])
            pltpu.async_copy(m_hbm.at[sl], rows[b], msem[b])

        def wait_fetch(b, ci):
            sl = pl.ds(base + ci * C, C)
            pltpu.make_async_copy(dst_hbm.at[sl], idx[b], isem[b]).wait()
            pltpu.make_async_copy(m_hbm.at[sl], rows[b], msem[b]).wait()

        for b in (0, 1):
            fetch(b, b)

        def zrow(i, _):
            for j in range(H // 16):
                zb[i, pl.ds(j * 16, 16)] = jnp.zeros((16,), jnp.float32)
            return 0

        lax.fori_loop(0, ZC, zrow, 0)

        def zcp(kk, _):
            pltpu.sync_copy(zb, shared.at[pl.ds(s * RPT + kk * ZC, ZC)])
            return 0

        lax.fori_loop(0, RPT // ZC, zcp, 0)
        plsc.subcore_barrier()

        def consume(b, ci, steady=True):
            wait_fetch(b, ci)
            pltpu.sync_copy(rows[b], shared.at[idx[b]], add=True)
            if steady:
                @pl.when(ci + 2 < NCH)
                def _():
                    fetch(b, ci + 2)

        def step(p, _):
            consume(0, 2 * p)
            consume(1, 2 * p + 1)
            return 0

        lax.fori_loop(0, NCH // 2, step, 0)
        if NCH % 2:
            consume((NCH - 1) % 2, NCH - 1, steady=False)
        plsc.subcore_barrier()

        def ocp(kk, _):
            r0 = s * RPT + kk * OC
            pltpu.sync_copy(shared.at[pl.ds(r0, OC)], ob)
            pltpu.sync_copy(ob, out_hbm.at[c, pl.ds(r0, OC)])
            return 0

        lax.fori_loop(0, RPT // OC, ocp, 0)

    return k


def _enc_body(x_ref, we_ref, be_ref, a_ref, b_ref, h_ref, hd_ref, hs_ref):
    h = jnp.dot(x_ref[...], we_ref[...],
                preferred_element_type=jnp.float32) + be_ref[...]
    h_ref[...] = h
    hd_ref[...] = jnp.dot(h, a_ref[...], preferred_element_type=jnp.float32)
    hs_ref[...] = jnp.dot(h, b_ref[...], preferred_element_type=jnp.float32)


def _edge_mlp_body(t_ref, w2_ref, b2_ref, m_ref):
    wu = lax.bitcast_convert_type(t_ref[...], jnp.uint32)
    lo = lax.bitcast_convert_type(wu << jnp.uint32(16), jnp.float32)
    hi = lax.bitcast_convert_type(wu & jnp.uint32(0xFFFF0000), jnp.float32)
    a = jnp.concatenate([lo, hi], axis=1)
    m = jnp.dot(a, w2_ref[...], preferred_element_type=jnp.float32) + b2_ref[...]
    m_ref[...] = jnp.maximum(m, 0.0)


def _norm(u, eps):
    mean = jnp.mean(u, axis=0, keepdims=True)
    var = jnp.mean(u * u, axis=0, keepdims=True) - mean * mean
    return (u - mean) * lax.rsqrt(var + eps)


def _update_body(h_ref, a0_ref, a1_ref, a2_ref, a3_ref,
                 u1a_ref, u1b_ref, bu1_ref,
                 u2_ref, bu2_ref, an_ref, bn_ref,
                 h_out, hd_out, hs_out, *, eps):
    h = h_ref[...]
    agg = (a0_ref[...] + a1_ref[...]) + (a2_ref[...] + a3_ref[...])
    u = jnp.dot(h, u1a_ref[...], preferred_element_type=jnp.float32)
    u = u + jnp.dot(agg, u1b_ref[...], preferred_element_type=jnp.float32)
    u = jnp.maximum(u + bu1_ref[...], 0.0)
    u = jnp.dot(u, u2_ref[...], preferred_element_type=jnp.float32) + bu2_ref[...]
    u = jnp.maximum(u, 0.0)
    hn = _norm(u, eps)
    h_out[...] = hn
    hd_out[...] = jnp.dot(hn, an_ref[...], preferred_element_type=jnp.float32)
    hs_out[...] = jnp.dot(hn, bn_ref[...], preferred_element_type=jnp.float32)


def _final_body(h_ref, a0_ref, a1_ref, a2_ref, a3_ref,
                u1a_ref, u1b_ref, bu1_ref,
                u2_ref, bu2_ref, wd_ref, bd_ref, x_out, *, eps):
    h = h_ref[...]
    agg = (a0_ref[...] + a1_ref[...]) + (a2_ref[...] + a3_ref[...])
    u = jnp.dot(h, u1a_ref[...], preferred_element_type=jnp.float32)
    u = u + jnp.dot(agg, u1b_ref[...], preferred_element_type=jnp.float32)
    u = jnp.maximum(u + bu1_ref[...], 0.0)
    u = jnp.dot(u, u2_ref[...], preferred_element_type=jnp.float32) + bu2_ref[...]
    u = jnp.maximum(u, 0.0)
    hn = _norm(u, eps)
    z = jnp.dot(hn, wd_ref[...], preferred_element_type=jnp.float32) + bd_ref[...]
    x_out[...] = 1.0 / (1.0 + jnp.exp(-z))


def _l2_body(x_ref, xt_ref, out_ref, *, ns, nb, dim):
    g = pl.program_id(0)
    xg = x_ref[0]
    xtg = xt_ref[0]
    acc = None
    p = None
    for k in range(dim):
        col = xg[:, k:k + 1]
        row = xtg[k:k + 1, :]
        f = 1.0 - jnp.maximum(col, row)
        acc = f if acc is None else acc * f
        pk = (1.0 - col * col) * 0.5
        p = pk if p is None else p * pk
    term3 = jnp.sum(acc) / (ns * ns)
    term2 = (2.0 / ns) * jnp.sum(p)
    term1 = (1.0 / 3.0) ** dim
    disc = jnp.sqrt(jnp.clip(term1 - term2 + term3, 1e-12, None))

    @pl.when(g == 0)
    def _():
        out_ref[...] = jnp.zeros_like(out_ref)

    out_ref[...] += jnp.full((1, 1), disc / nb, jnp.float32)


def _const_spec(shape):
    return pl.BlockSpec(shape, lambda *_: tuple(0 for _ in shape))


def kernel(x, params, edge_index, batch):
    n, dim = x.shape
    e = edge_index.shape[1]
    nb = 10
    ns = n // nb
    eps = 1e-5

    src = edge_index[0]
    dst = edge_index[1]

    layers = params['layers']
    nlayers = len(layers)
    _q = np.arange(H)
    _qh = _q % (H // 2)
    _perm = 32 * (_qh // 16) + (_qh % 16) + 16 * (_q // (H // 2))
    m1a = [lp['m1'][0][:H] for lp in layers]
    m1b = [lp['m1'][0][H:] for lp in layers]
    b1 = [lp['m1'][1] for lp in layers]
    w2 = [lp['m2'][0][_perm] for lp in layers]
    b2 = [lp['m2'][1].reshape(1, H) for lp in layers]
    u1a = [lp['u1'][0][:H] for lp in layers]
    u1b = [lp['u1'][0][H:] for lp in layers]
    bu1 = [lp['u1'][1].reshape(1, H) for lp in layers]
    u2 = [lp['u2'][0] for lp in layers]
    bu2 = [lp['u2'][1].reshape(1, H) for lp in layers]
    we, be = params['enc']
    wd, bd = params['dec']
    be = be.reshape(1, H)
    bd = bd.reshape(1, dim)

    f32 = jnp.float32

    h, hd, hs = pl.pallas_call(
        _enc_body,
        out_shape=[jax.ShapeDtypeStruct((n, H), f32)] * 3,
    )(x, we, be, m1a[0], m1b[0])

    npad = ((n + NSUB * 40 - 1) // (NSUB * 40)) * (NSUB * 40)
    e2 = e // 2
    gather = _sc_gather_add(e2, n)
    scatter = _sc_scatter_add(e2, npad)
    dsts = (dst[:e2], dst[e2:])
    srcs = (src[:e2], src[e2:])

    BE = 4000
    edge_mlp = pl.pallas_call(
        _edge_mlp_body,
        grid=(e2 // BE,),
        in_specs=[
            pl.BlockSpec((BE, H // 2), lambda i: (i, 0)),
            _const_spec((H, H)),
            _const_spec((1, H)),
        ],
        out_specs=pl.BlockSpec((BE, H), lambda i: (i, 0)),
        out_shape=jax.ShapeDtypeStruct((e2, H), f32),
    )

    node_specs = [
        pl.BlockSpec((ns, H), lambda g: (g, 0)),
        pl.BlockSpec((ns, H), lambda g: (g, 0)),
        pl.BlockSpec((ns, H), lambda g: (g, 0)),
        pl.BlockSpec((ns, H), lambda g: (g, 0)),
        pl.BlockSpec((ns, H), lambda g: (g, 0)),
        _const_spec((H, H)), _const_spec((H, H)), _const_spec((1, H)),
        _const_spec((H, H)), _const_spec((1, H)),
    ]

    for l in range(nlayers):
        t2a = gather(hd, hs, dsts[0], srcs[0], b1[l])
        t2b = gather(hd, hs, dsts[1], srcs[1], b1[l])
        ma = edge_mlp(t2a, w2[l], b2[l])
        mb = edge_mlp(t2b, w2[l], b2[l])
        pa = scatter(ma, dsts[0])
        pb = scatter(mb, dsts[1])
        if l + 1 < nlayers:
            h, hd, hs = pl.pallas_call(
                functools.partial(_update_body, eps=eps),
                grid=(nb,),
                in_specs=node_specs + [_const_spec((H, H)), _const_spec((H, H))],
                out_specs=[pl.BlockSpec((ns, H), lambda g: (g, 0))] * 3,
                out_shape=[jax.ShapeDtypeStruct((n, H), f32)] * 3,
            )(h, pa[0], pa[1], pb[0], pb[1],
              u1a[l], u1b[l], bu1[l], u2[l], bu2[l],
              m1a[l + 1], m1b[l + 1])
        else:
            xo = pl.pallas_call(
                functools.partial(_final_body, eps=eps),
                grid=(nb,),
                in_specs=node_specs + [_const_spec((H, dim)),
                                       _const_spec((1, dim))],
                out_specs=pl.BlockSpec((ns, dim), lambda g: (g, 0)),
                out_shape=jax.ShapeDtypeStruct((n, dim), f32),
            )(h, pa[0], pa[1], pb[0], pb[1],
              u1a[l], u1b[l], bu1[l], u2[l], bu2[l],
              wd, bd)

    X = xo.reshape(nb, ns, dim)
    Xt = jnp.swapaxes(X, 1, 2)

    loss2d = pl.pallas_call(
        functools.partial(_l2_body, ns=ns, nb=nb, dim=dim),
        grid=(nb,),
        in_specs=[
            pl.BlockSpec((1, ns, dim), lambda g: (g, 0, 0)),
            pl.BlockSpec((1, dim, ns), lambda g: (g, 0, 0)),
        ],
        out_specs=pl.BlockSpec((1, 1), lambda g: (0, 0)),
        out_shape=jax.ShapeDtypeStruct((1, 1), f32),
    )(X, Xt)
    loss = loss2d[0, 0]
    return (loss, X)


# speedup vs baseline: 1.0351x; 1.0351x over previous
import functools

import numpy as np

import jax
import jax.numpy as jnp
from jax import lax
from jax.experimental import pallas as pl
from jax.experimental.pallas import tpu as pltpu
from jax.experimental.pallas import tpu_sc as plsc

H = 128
NC = 2
NSUB = 16
NW = NC * NSUB


def _sc_mesh():
    return plsc.VectorSubcoreMesh(core_axis_name="c", subcore_axis_name="s",
                                  num_cores=NC, num_subcores=NSUB)


@functools.lru_cache(maxsize=None)
def _sc_gather_add(E, N):
    EPW = E // NW
    C = 40
    NCH = EPW // C
    NBUF = 3
    SCH = 80
    NSTG = N // SCH
    assert EPW % C == 0 and N % SCH == 0

    @functools.partial(
        pl.kernel,
        out_type=jax.ShapeDtypeStruct((E, H // 2), jnp.int32),
        mesh=_sc_mesh(),
        scratch_types=[
            [pltpu.VMEM((C,), jnp.int32)] * NBUF,
            [pltpu.VMEM((C,), jnp.int32)] * NBUF,
            [pltpu.VMEM((C, H), jnp.float32)] * NBUF,
            [pltpu.VMEM((C, H), jnp.float32)] * NBUF,
            [pltpu.VMEM((C, H // 2), jnp.int32)] * NBUF,
            pltpu.VMEM((H,), jnp.float32),
            pltpu.VMEM_SHARED((N, H), jnp.float32),
            [pltpu.SemaphoreType.DMA] * NBUF,
            [pltpu.SemaphoreType.DMA] * NBUF,
            [pltpu.SemaphoreType.DMA] * NBUF,
            [pltpu.SemaphoreType.DMA] * NBUF,
            pltpu.SemaphoreType.DMA,
        ],
    )
    def k(hd_hbm, hs_hbm, dst_hbm, src_hbm, b1_hbm, out_hbm,
          idxl, idxr, rl, rr, tb, b1v, shared, gl, gr, osem, isem, ssem):
        c = lax.axis_index("c")
        s = lax.axis_index("s")
        base = (c * NSUB + s) * EPW
        pltpu.sync_copy(b1_hbm, b1v)

        def stage(tab_hbm):
            def body(kk, _):
                ch = s + kk * NSUB

                @pl.when(ch < NSTG)
                def _():
                    sl = pl.ds(ch * SCH, SCH)
                    pltpu.async_copy(tab_hbm.at[sl], shared.at[sl], ssem).wait()
                return 0

            lax.fori_loop(0, (NSTG + NSUB - 1) // NSUB, body, 0)

        def run(lidx_hbm, rem_hbm, ridx_hbm):
            def fetch_idx(b, ci):
                sl = pl.ds(base + ci * C, C)
                pltpu.async_copy(lidx_hbm.at[sl], idxl[b], isem[b])
                pltpu.async_copy(ridx_hbm.at[sl], idxr[b], isem[b])

            def wait_idx(b, ci):
                sl = pl.ds(base + ci * C, C)
                pltpu.make_async_copy(lidx_hbm.at[sl], idxl[b], isem[b]).wait()
                pltpu.make_async_copy(ridx_hbm.at[sl], idxr[b], isem[b]).wait()

            def fire(b):
                pltpu.async_copy(shared.at[idxl[b]], rl[b], gl[b])
                pltpu.async_copy(rem_hbm.at[idxr[b]], rr[b], gr[b])

            def wait_gather(b):
                pltpu.make_async_copy(shared.at[idxl[b]], rl[b], gl[b]).wait()
                pltpu.make_async_copy(rem_hbm.at[idxr[b]], rr[b], gr[b]).wait()

            def wait_out(b, ci):
                sl = pl.ds(base + ci * C, C)
                pltpu.make_async_copy(tb[b], out_hbm.at[sl], osem[b]).wait()

            def rhu_pack(x, y):
                ux = lax.bitcast_convert_type(x, jnp.uint32)
                uy = lax.bitcast_convert_type(y, jnp.uint32)
                w = (((ux + jnp.uint32(0x8000)) >> 16)
                     | ((uy + jnp.uint32(0x8000)) & jnp.uint32(0xFFFF0000)))
                return lax.bitcast_convert_type(w, jnp.int32)

            bias = [b1v[pl.ds(16 * j, 16)] for j in range(H // 16)]

            def consume(b, ci, steady=True):
                wait_gather(b)
                if steady:
                    @pl.when(ci + NBUF < NCH)
                    def _():
                        fetch_idx(b, ci + NBUF)

                @pl.when(ci >= NBUF)
                def _():
                    wait_out(b, ci - NBUF)

                def addrow(i, _):
                    for j in range(H // 32):
                        slx = pl.ds(32 * j, 16)
                        sly = pl.ds(32 * j + 16, 16)
                        x = jnp.maximum(rl[b][i, slx] + rr[b][i, slx]
                                        + bias[2 * j], 0.0)
                        y = jnp.maximum(rl[b][i, sly] + rr[b][i, sly]
                                        + bias[2 * j + 1], 0.0)
                        tb[b][i, pl.ds(16 * j, 16)] = rhu_pack(x, y)
                    return 0

                lax.fori_loop(0, C, addrow, 0)
                pltpu.async_copy(tb[b], out_hbm.at[pl.ds(base + ci * C, C)],
                                 osem[b])
                if steady:
                    @pl.when(ci + NBUF < NCH)
                    def _():
                        wait_idx(b, ci + NBUF)
                        fire(b)

            for b in range(NBUF):
                fetch_idx(b, b)
                wait_idx(b, b)
                fire(b)

            def step(p, _):
                for b in range(NBUF):
                    consume(b, NBUF * p + b)
                return 0

            lax.fori_loop(0, NCH // NBUF, step, 0)
            for ci in range((NCH // NBUF) * NBUF, NCH):
                consume(ci % NBUF, ci, steady=False)
            for ci in range(NCH - NBUF, NCH):
                wait_out(ci % NBUF, ci)

        @pl.when(c == 0)
        def _():
            stage(hd_hbm)

        @pl.when(c == 1)
        def _():
            stage(hs_hbm)

        plsc.subcore_barrier()

        @pl.when(c == 0)
        def _():
            run(dst_hbm, hs_hbm, src_hbm)

        @pl.when(c == 1)
        def _():
            run(src_hbm, hd_hbm, dst_hbm)

    return k


@functools.lru_cache(maxsize=None)
def _sc_scatter_add(E, NPAD):
    EPW = E // NW
    C = 80
    RPT = NPAD // NSUB
    ZC = 40
    OC = 128
    assert EPW % C == 0 and RPT % ZC == 0 and RPT % OC == 0 and RPT % 8 == 0

    @functools.partial(
        pl.kernel,
        out_type=jax.ShapeDtypeStruct((NC, NPAD, H), jnp.float32),
        mesh=_sc_mesh(),
        scratch_types=[
            [pltpu.VMEM((C,), jnp.int32)] * 2,
            [pltpu.VMEM((C, H), jnp.float32)] * 2,
            pltpu.VMEM((ZC, H), jnp.float32),
            pltpu.VMEM((OC, H), jnp.float32),
            pltpu.VMEM_SHARED((NPAD, H), jnp.float32),
            [pltpu.SemaphoreType.DMA] * 2,
            [pltpu.SemaphoreType.DMA] * 2,
        ],
    )
    def k(m_hbm, dst_hbm, out_hbm, idx, rows, zb, ob, shared, msem, isem):
        c = lax.axis_index("c")
        s = lax.axis_index("s")
        wid = c * NSUB + s
        base = wid * EPW
        NCH = EPW // C

        def fetch(b, ci):
            sl = pl.ds(base + ci * C, C)
            pltpu.async_copy(dst_hbm.at[sl], idx[b], isem[b])
            pltpu.async_copy(m_hbm.at[sl], rows[b], msem[b])

        def wait_fetch(b, ci):
            sl = pl.ds(base + ci * C, C)
            pltpu.make_async_copy(dst_hbm.at[sl], idx[b], isem[b]).wait()
            pltpu.make_async_copy(m_hbm.at[sl], rows[b], msem[b]).wait()

        for b in (0, 1):
            fetch(b, b)

        def zrow(i, _):
            for j in range(H // 16):
                zb[i, pl.ds(j * 16, 16)] = jnp.zeros((16,), jnp.float32)
            return 0

        lax.fori_loop(0, ZC, zrow, 0)

        def zcp(kk, _):
            pltpu.sync_copy(zb, shared.at[pl.ds(s * RPT + kk * ZC, ZC)])
            return 0

        lax.fori_loop(0, RPT // ZC, zcp, 0)
        plsc.subcore_barrier()

        def consume(b, ci, steady=True):
            wait_fetch(b, ci)
            pltpu.sync_copy(rows[b], shared.at[idx[b]], add=True)
            if steady:
                @pl.when(ci + 2 < NCH)
                def _():
                    fetch(b, ci + 2)

        def step(p, _):
            consume(0, 2 * p)
            consume(1, 2 * p + 1)
            return 0

        lax.fori_loop(0, NCH // 2, step, 0)
        if NCH % 2:
            consume((NCH - 1) % 2, NCH - 1, steady=False)
        plsc.subcore_barrier()

        def ocp(kk, _):
            r0 = s * RPT + kk * OC
            pltpu.sync_copy(shared.at[pl.ds(r0, OC)], ob)
            pltpu.sync_copy(ob, out_hbm.at[c, pl.ds(r0, OC)])
            return 0

        lax.fori_loop(0, RPT // OC, ocp, 0)

    return k


def _enc_body(x_ref, we_ref, be_ref, a_ref, b_ref, h_ref, hd_ref, hs_ref):
    h = jnp.dot(x_ref[...], we_ref[...],
                preferred_element_type=jnp.float32) + be_ref[...]
    h_ref[...] = h
    hd_ref[...] = jnp.dot(h, a_ref[...], preferred_element_type=jnp.float32)
    hs_ref[...] = jnp.dot(h, b_ref[...], preferred_element_type=jnp.float32)


def _edge_mlp_body(t_ref, w2_ref, b2_ref, m_ref):
    wu = lax.bitcast_convert_type(t_ref[...], jnp.uint32)
    lo = lax.bitcast_convert_type(wu << jnp.uint32(16), jnp.float32)
    hi = lax.bitcast_convert_type(wu & jnp.uint32(0xFFFF0000), jnp.float32)
    a = jnp.concatenate([lo, hi], axis=1)
    m = jnp.dot(a, w2_ref[...], preferred_element_type=jnp.float32) + b2_ref[...]
    m_ref[...] = jnp.maximum(m, 0.0)


def _norm(u, eps):
    mean = jnp.mean(u, axis=0, keepdims=True)
    var = jnp.mean(u * u, axis=0, keepdims=True) - mean * mean
    return (u - mean) * lax.rsqrt(var + eps)


def _update_body(h_ref, a0_ref, a1_ref, u1a_ref, u1b_ref, bu1_ref,
                 u2_ref, bu2_ref, an_ref, bn_ref,
                 h_out, hd_out, hs_out, *, eps):
    h = h_ref[...]
    agg = a0_ref[...] + a1_ref[...]
    u = jnp.dot(h, u1a_ref[...], preferred_element_type=jnp.float32)
    u = u + jnp.dot(agg, u1b_ref[...], preferred_element_type=jnp.float32)
    u = jnp.maximum(u + bu1_ref[...], 0.0)
    u = jnp.dot(u, u2_ref[...], preferred_element_type=jnp.float32) + bu2_ref[...]
    u = jnp.maximum(u, 0.0)
    hn = _norm(u, eps)
    h_out[...] = hn
    hd_out[...] = jnp.dot(hn, an_ref[...], preferred_element_type=jnp.float32)
    hs_out[...] = jnp.dot(hn, bn_ref[...], preferred_element_type=jnp.float32)


def _final_body(h_ref, a0_ref, a1_ref, u1a_ref, u1b_ref, bu1_ref,
                u2_ref, bu2_ref, wd_ref, bd_ref, x_out, *, eps):
    h = h_ref[...]
    agg = a0_ref[...] + a1_ref[...]
    u = jnp.dot(h, u1a_ref[...], preferred_element_type=jnp.float32)
    u = u + jnp.dot(agg, u1b_ref[...], preferred_element_type=jnp.float32)
    u = jnp.maximum(u + bu1_ref[...], 0.0)
    u = jnp.dot(u, u2_ref[...], preferred_element_type=jnp.float32) + bu2_ref[...]
    u = jnp.maximum(u, 0.0)
    hn = _norm(u, eps)
    z = jnp.dot(hn, wd_ref[...], preferred_element_type=jnp.float32) + bd_ref[...]
    x_out[...] = 1.0 / (1.0 + jnp.exp(-z))


def _l2_body(x_ref, xt_ref, out_ref, *, ns, nb, dim):
    g = pl.program_id(0)
    xg = x_ref[0]
    xtg = xt_ref[0]
    acc = None
    p = None
    for k in range(dim):
        col = xg[:, k:k + 1]
        row = xtg[k:k + 1, :]
        f = 1.0 - jnp.maximum(col, row)
        acc = f if acc is None else acc * f
        pk = (1.0 - col * col) * 0.5
        p = pk if p is None else p * pk
    term3 = jnp.sum(acc) / (ns * ns)
    term2 = (2.0 / ns) * jnp.sum(p)
    term1 = (1.0 / 3.0) ** dim
    disc = jnp.sqrt(jnp.clip(term1 - term2 + term3, 1e-12, None))

    @pl.when(g == 0)
    def _():
        out_ref[...] = jnp.zeros_like(out_ref)

    out_ref[...] += jnp.full((1, 1), disc / nb, jnp.float32)


def _const_spec(shape):
    return pl.BlockSpec(shape, lambda *_: tuple(0 for _ in shape))


def kernel(x, params, edge_index, batch):
    n, dim = x.shape
    e = edge_index.shape[1]
    nb = 10
    ns = n // nb
    eps = 1e-5

    src = edge_index[0]
    dst = edge_index[1]

    layers = params['layers']
    nlayers = len(layers)
    _q = np.arange(H)
    _qh = _q % (H // 2)
    _perm = 32 * (_qh // 16) + (_qh % 16) + 16 * (_q // (H // 2))
    m1a = [lp['m1'][0][:H] for lp in layers]
    m1b = [lp['m1'][0][H:] for lp in layers]
    b1 = [lp['m1'][1] for lp in layers]
    w2 = [lp['m2'][0][_perm] for lp in layers]
    b2 = [lp['m2'][1].reshape(1, H) for lp in layers]
    u1a = [lp['u1'][0][:H] for lp in layers]
    u1b = [lp['u1'][0][H:] for lp in layers]
    bu1 = [lp['u1'][1].reshape(1, H) for lp in layers]
    u2 = [lp['u2'][0] for lp in layers]
    bu2 = [lp['u2'][1].reshape(1, H) for lp in layers]
    we, be = params['enc']
    wd, bd = params['dec']
    be = be.reshape(1, H)
    bd = bd.reshape(1, dim)

    f32 = jnp.float32

    h, hd, hs = pl.pallas_call(
        _enc_body,
        out_shape=[jax.ShapeDtypeStruct((n, H), f32)] * 3,
    )(x, we, be, m1a[0], m1b[0])

    npad = ((n + NSUB * 40 - 1) // (NSUB * 40)) * (NSUB * 40)
    gather = _sc_gather_add(e, n)
    scatter = _sc_scatter_add(e, npad)

    BE = 4000
    edge_mlp = pl.pallas_call(
        _edge_mlp_body,
        grid=(e // BE,),
        in_specs=[
            pl.BlockSpec((BE, H // 2), lambda i: (i, 0)),
            _const_spec((H, H)),
            _const_spec((1, H)),
        ],
        out_specs=pl.BlockSpec((BE, H), lambda i: (i, 0)),
        out_shape=jax.ShapeDtypeStruct((e, H), f32),
    )

    node_specs = [
        pl.BlockSpec((ns, H), lambda g: (g, 0)),
        pl.BlockSpec((ns, H), lambda g: (g, 0)),
        pl.BlockSpec((ns, H), lambda g: (g, 0)),
        _const_spec((H, H)), _const_spec((H, H)), _const_spec((1, H)),
        _const_spec((H, H)), _const_spec((1, H)),
    ]

    for l in range(nlayers):
        t2 = gather(hd, hs, dst, src, b1[l])
        m = edge_mlp(t2, w2[l], b2[l])
        aggp = scatter(m, dst)
        if l + 1 < nlayers:
            h, hd, hs = pl.pallas_call(
                functools.partial(_update_body, eps=eps),
                grid=(nb,),
                in_specs=node_specs + [_const_spec((H, H)), _const_spec((H, H))],
                out_specs=[pl.BlockSpec((ns, H), lambda g: (g, 0))] * 3,
                out_shape=[jax.ShapeDtypeStruct((n, H), f32)] * 3,
            )(h, aggp[0], aggp[1], u1a[l], u1b[l], bu1[l], u2[l], bu2[l],
              m1a[l + 1], m1b[l + 1])
        else:
            xo = pl.pallas_call(
                functools.partial(_final_body, eps=eps),
                grid=(nb,),
                in_specs=node_specs + [_const_spec((H, dim)),
                                       _const_spec((1, dim))],
                out_specs=pl.BlockSpec((ns, dim), lambda g: (g, 0)),
                out_shape=jax.ShapeDtypeStruct((n, dim), f32),
            )(h, aggp[0], aggp[1], u1a[l], u1b[l], bu1[l], u2[l], bu2[l],
              wd, bd)

    X = xo.reshape(nb, ns, dim)
    Xt = jnp.swapaxes(X, 1, 2)

    loss2d = pl.pallas_call(
        functools.partial(_l2_body, ns=ns, nb=nb, dim=dim),
        grid=(nb,),
        in_specs=[
            pl.BlockSpec((1, ns, dim), lambda g: (g, 0, 0)),
            pl.BlockSpec((1, dim, ns), lambda g: (g, 0, 0)),
        ],
        out_specs=pl.BlockSpec((1, 1), lambda g: (0, 0)),
        out_shape=jax.ShapeDtypeStruct((1, 1), f32),
    )(X, Xt)
    loss = loss2d[0, 0]
    return (loss, X)
